# baseline (speedup 1.0000x reference)
import functools

import jax
import jax.numpy as jnp
from jax import lax
from jax.experimental import pallas as pl
from jax.experimental.pallas import tpu as pltpu
from jax.experimental.pallas import tpu_sc as plsc

N = 10000
E = 320000
D = 128
NP = 10240
NC = 2
NS = 16
NW = NC * NS
EW = E // NW
C = 128
PADW = 240
NCHUNK = (EW + PADW) // C
CB = 20
NB = NCHUNK // CB
DEGW = NP // NS
NA = 10240
ROWS_PER_TILE = NA // NS

_mesh = plsc.VectorSubcoreMesh(core_axis_name="c", subcore_axis_name="s")


def _deg_body(ei_hbm, ones_hbm, zz_hbm, out_hbm, idx_v, ones_v, sem, acc_sh):
    c = lax.axis_index("c")
    s = lax.axis_index("s")
    w = c * NS + s
    pltpu.sync_copy(zz_hbm, acc_sh.at[pl.ds(s * DEGW, DEGW)])
    pltpu.sync_copy(ones_hbm, ones_v)
    pltpu.sync_copy(ei_hbm.at[1, w], idx_v)
    plsc.subcore_barrier()

    @pl.loop(0, NB)
    def _(b):
        @pl.loop(0, CB)
        def _(i):
            pltpu.async_copy(ones_v, acc_sh.at[idx_v.at[b, i]], sem, add=True)

    @pl.loop(0, NB * CB)
    def _(i):
        pltpu.make_async_copy(ones_v, acc_sh.at[idx_v.at[0, 0]], sem).wait()

    plsc.subcore_barrier()
    pltpu.sync_copy(acc_sh.at[pl.ds(s * DEGW, DEGW)],
                    out_hbm.at[c, pl.ds(s * DEGW, DEGW)])


def _degree(ei_deg, ones_c, zz_deg):
    k = pl.kernel(
        _deg_body,
        out_type=jax.ShapeDtypeStruct((NC, NP), jnp.float32),
        mesh=_mesh,
        scratch_types=[
            pltpu.VMEM((NB, CB, C), jnp.int32),
            pltpu.VMEM((C,), jnp.float32),
            pltpu.SemaphoreType.DMA,
            pltpu.VMEM_SHARED((NP,), jnp.float32),
        ],
    )
    return k(ei_deg, ones_c, zz_deg)


def _prop_body(g_hbm, ei_hbm, zz_hbm, out_hbm,
               idxs_v, idxd_v, rows0_v, rows1_v,
               sem0, sem1, sem2, sem3, acc_sh):
    c = lax.axis_index("c")
    s = lax.axis_index("s")
    w = c * NS + s
    pltpu.sync_copy(zz_hbm, acc_sh.at[pl.ds(s * ROWS_PER_TILE, ROWS_PER_TILE), :])
    plsc.subcore_barrier()

    rows = (rows0_v, rows1_v)
    gsem = (sem0, sem1)
    ssem = (sem2, sem3)

    def _gather(j, b):
        pltpu.async_copy(g_hbm.at[idxs_v.at[j]], rows[b], gsem[b])

    def _wait_gather(b):
        pltpu.make_async_copy(g_hbm.at[idxs_v.at[0]], rows[b], gsem[b]).wait()

    def _scatter(j, b):
        pltpu.async_copy(rows[b], acc_sh.at[idxd_v.at[j]], ssem[b], add=True)

    def _wait_scatter(b):
        pltpu.make_async_copy(rows[b], acc_sh.at[idxd_v.at[0]], ssem[b]).wait()

    def _step(j, b):
        _wait_scatter(b)
        _gather(j, b)
        _wait_gather(1 - b)
        _scatter(j - 1, 1 - b)

    @pl.loop(0, NB)
    def _(b):
        pltpu.async_copy(ei_hbm.at[0, w, b], idxs_v, sem0)
        pltpu.async_copy(ei_hbm.at[1, w, b], idxd_v, sem1)
        pltpu.make_async_copy(ei_hbm.at[0, w, b], idxs_v, sem0).wait()
        pltpu.make_async_copy(ei_hbm.at[1, w, b], idxd_v, sem1).wait()
        _gather(0, 0)
        _gather(1, 1)
        _wait_gather(0)
        _scatter(0, 0)

        @pl.loop(1, 10)
        def _(m):
            j = 2 * m
            _step(j + 0, 0)
            _step(j + 1, 1)

        _wait_gather(1)
        _scatter(19, 1)
        _wait_scatter(0)
        _wait_scatter(1)

    plsc.subcore_barrier()
    pltpu.sync_copy(acc_sh.at[pl.ds(s * ROWS_PER_TILE, ROWS_PER_TILE), :],
                    out_hbm.at[c, pl.ds(s * ROWS_PER_TILE, ROWS_PER_TILE), :])


def _propagate(g, ei_prop, zz_rows):
    k = pl.kernel(
        _prop_body,
        out_type=jax.ShapeDtypeStruct((NC, NA, D), jnp.float32),
        mesh=_mesh,
        scratch_types=[
            pltpu.VMEM((CB, C), jnp.int32),
            pltpu.VMEM((CB, C), jnp.int32),
            pltpu.VMEM((C, D), jnp.float32),
            pltpu.VMEM((C, D), jnp.float32),
            pltpu.SemaphoreType.DMA,
            pltpu.SemaphoreType.DMA,
            pltpu.SemaphoreType.DMA,
            pltpu.SemaphoreType.DMA,
            pltpu.VMEM_SHARED((NA, D), jnp.float32),
        ],
    )
    return k(g, ei_prop, zz_rows)


_R = 2000


def _mm1_body(d0_ref, d1_ref, x_ref, w_ref, g_ref, dis_ref):
    deg = d0_ref[...] + d1_ref[...] + 1.0
    dis = lax.rsqrt(deg)
    h = jnp.dot(x_ref[...], w_ref[...], preferred_element_type=jnp.float32)
    g_ref[...] = h * dis
    dis_ref[...] = dis


def _mid_body(s_ref, g1_ref, dis_ref, b_ref, w_ref, g2_ref):
    dis = dis_ref[...]
    t = (s_ref[0] + s_ref[1] + g1_ref[...]) * dis + b_ref[...]
    t = jnp.maximum(t, 0.0)
    g2_ref[...] = jnp.dot(t, w_ref[...], preferred_element_type=jnp.float32) * dis


def _fin_body(s_ref, g2_ref, dis_ref, b_ref, o_ref):
    o_ref[...] = ((s_ref[0] + s_ref[1] + g2_ref[...]) * dis_ref[...]
                  + b_ref[...])


def _row_spec(width):
    return pl.BlockSpec((_R, width), lambda i: (i, 0))


def _s_spec():
    return pl.BlockSpec((NC, _R, D), lambda i: (0, i, 0))


def _full_spec(shape):
    return pl.BlockSpec(shape, lambda i: (0,) * len(shape))


def _mm1(d0, d1, x, W1):
    return pl.pallas_call(
        _mm1_body,
        grid=(N // _R,),
        in_specs=[_row_spec(1), _row_spec(1), _row_spec(D), _full_spec((D, D))],
        out_specs=[_row_spec(D), _row_spec(1)],
        out_shape=[jax.ShapeDtypeStruct((N, D), jnp.float32),
                   jax.ShapeDtypeStruct((N, 1), jnp.float32)],
    )(d0, d1, x, W1)


def _mid(s, g1, dis, b1, W2):
    return pl.pallas_call(
        _mid_body,
        grid=(N // _R,),
        in_specs=[_s_spec(), _row_spec(D), _row_spec(1),
                  _full_spec((1, D)), _full_spec((D, D))],
        out_specs=_row_spec(D),
        out_shape=jax.ShapeDtypeStruct((N, D), jnp.float32),
    )(s, g1, dis, b1, W2)


def _fin(s, g2, dis, b2):
    return pl.pallas_call(
        _fin_body,
        grid=(N // _R,),
        in_specs=[_s_spec(), _row_spec(D), _row_spec(1), _full_spec((1, D))],
        out_specs=_row_spec(D),
        out_shape=jax.ShapeDtypeStruct((N, D), jnp.float32),
    )(s, g2, dis, b2)


def kernel(x, edge_index, W1, b1, W2, b2):
    ei3 = edge_index.reshape(2, NW, EW)
    wk = jnp.arange(NW, dtype=jnp.int32)[:, None]
    pk = jnp.arange(PADW, dtype=jnp.int32)[None, :]
    dsrc = (wk * 977 + pk * 131) % N
    ddst = N + (wk * 37 + pk) % (NA - N)
    dummy = jnp.stack([dsrc, ddst])
    ei_prop = jnp.concatenate([ei3, dummy], axis=2).reshape(2, NW, NB, CB, C)
    ones_c = jnp.ones((C,), jnp.float32)
    zz_deg = jnp.zeros((DEGW,), jnp.float32)
    zz_rows = jnp.zeros((ROWS_PER_TILE, D), jnp.float32)

    deg = _degree(ei_prop, ones_c, zz_deg)
    g1, dis = _mm1(deg[0, :N, None], deg[1, :N, None], x, W1)

    s1 = _propagate(g1, ei_prop, zz_rows)
    g2 = _mid(s1, g1, dis, b1[None, :], W2)

    s2 = _propagate(g2, ei_prop, zz_rows)
    return _fin(s2, g2, dis, b2[None, :])


# speedup vs baseline: 1.0710x; 1.0710x over previous
import functools

import jax
import jax.numpy as jnp
from jax import lax
from jax.experimental import pallas as pl
from jax.experimental.pallas import tpu as pltpu
from jax.experimental.pallas import tpu_sc as plsc

N = 10000
E = 320000
D = 128
NP = 10240
NC = 2
NS = 16
NW = NC * NS
EW = E // NW
C = 112
PADW = 80
NCHUNK = (EW + PADW) // C
CB = 18
NB = NCHUNK // CB
DEGW = NP // NS
NA = 10112
ROWS_PER_TILE = NA // NS

_mesh = plsc.VectorSubcoreMesh(core_axis_name="c", subcore_axis_name="s")


def _deg_body(ei_hbm, ones_hbm, zz_hbm, out_hbm, idx_v, ones_v, sem, acc_sh):
    c = lax.axis_index("c")
    s = lax.axis_index("s")
    w = c * NS + s
    pltpu.sync_copy(zz_hbm, acc_sh.at[pl.ds(s * DEGW, DEGW)])
    pltpu.sync_copy(ones_hbm, ones_v)
    pltpu.sync_copy(ei_hbm.at[1, w], idx_v)
    plsc.subcore_barrier()

    @pl.loop(0, NB)
    def _(b):
        @pl.loop(0, CB)
        def _(i):
            pltpu.async_copy(ones_v, acc_sh.at[idx_v.at[b, i]], sem, add=True)

    @pl.loop(0, NB * CB)
    def _(i):
        pltpu.make_async_copy(ones_v, acc_sh.at[idx_v.at[0, 0]], sem).wait()

    plsc.subcore_barrier()
    pltpu.sync_copy(acc_sh.at[pl.ds(s * DEGW, DEGW)],
                    out_hbm.at[c, pl.ds(s * DEGW, DEGW)])


def _degree(ei_deg, ones_c, zz_deg):
    k = pl.kernel(
        _deg_body,
        out_type=jax.ShapeDtypeStruct((NC, NP), jnp.float32),
        mesh=_mesh,
        scratch_types=[
            pltpu.VMEM((NB, CB, C), jnp.int32),
            pltpu.VMEM((C,), jnp.float32),
            pltpu.SemaphoreType.DMA,
            pltpu.VMEM_SHARED((NP,), jnp.float32),
        ],
    )
    return k(ei_deg, ones_c, zz_deg)


def _prop_body(g_hbm, ei_hbm, zz_hbm, out_hbm,
               idxs_v, idxd_v, rows0_v, rows1_v, rows2_v,
               sem0, sem1, sem2, sem3, sem4, sem5, semz, acc_sh):
    c = lax.axis_index("c")
    s = lax.axis_index("s")
    w = c * NS + s
    zslice = acc_sh.at[pl.ds(s * ROWS_PER_TILE, ROWS_PER_TILE), :]
    pltpu.async_copy(zz_hbm, zslice, semz)

    rows = (rows0_v, rows1_v, rows2_v)
    gsem = (sem0, sem1, sem2)
    ssem = (sem3, sem4, sem5)

    def _gather(j, b):
        pltpu.async_copy(g_hbm.at[idxs_v.at[j]], rows[b], gsem[b])

    def _wait_gather(b):
        pltpu.make_async_copy(g_hbm.at[idxs_v.at[0]], rows[b], gsem[b]).wait()

    def _scatter(j, b):
        pltpu.async_copy(rows[b], acc_sh.at[idxd_v.at[j]], ssem[b], add=True)

    def _wait_scatter(b):
        pltpu.make_async_copy(rows[b], acc_sh.at[idxd_v.at[0]], ssem[b]).wait()

    def _step(j, b):
        _wait_scatter(b)
        _gather(j, b)
        _wait_gather((b + 1) % 3)
        _scatter(j - 2, (b + 1) % 3)

    @pl.loop(0, NB)
    def _(b):
        pltpu.async_copy(ei_hbm.at[0, w, b], idxs_v, sem0)
        pltpu.async_copy(ei_hbm.at[1, w, b], idxd_v, sem1)
        pltpu.make_async_copy(ei_hbm.at[0, w, b], idxs_v, sem0).wait()
        pltpu.make_async_copy(ei_hbm.at[1, w, b], idxd_v, sem1).wait()
        _gather(0, 0)
        _gather(1, 1)
        _gather(2, 2)

        @pl.when(b == 0)
        def _():
            pltpu.make_async_copy(zz_hbm, zslice, semz).wait()
            plsc.subcore_barrier()

        _wait_gather(0)
        _scatter(0, 0)

        @pl.loop(1, 6)
        def _(m):
            j = 3 * m
            _step(j + 0, 0)
            _step(j + 1, 1)
            _step(j + 2, 2)

        _wait_gather(1)
        _scatter(16, 1)
        _wait_gather(2)
        _scatter(17, 2)
        _wait_scatter(0)
        _wait_scatter(1)
        _wait_scatter(2)

    plsc.subcore_barrier()
    pltpu.sync_copy(acc_sh.at[pl.ds(s * ROWS_PER_TILE, ROWS_PER_TILE), :],
                    out_hbm.at[c, pl.ds(s * ROWS_PER_TILE, ROWS_PER_TILE), :])


def _propagate(g, ei_prop, zz_rows):
    k = pl.kernel(
        _prop_body,
        out_type=jax.ShapeDtypeStruct((NC, NA, D), jnp.float32),
        mesh=_mesh,
        scratch_types=[
            pltpu.VMEM((CB, C), jnp.int32),
            pltpu.VMEM((CB, C), jnp.int32),
            pltpu.VMEM((C, D), jnp.float32),
            pltpu.VMEM((C, D), jnp.float32),
            pltpu.VMEM((C, D), jnp.float32),
            pltpu.SemaphoreType.DMA,
            pltpu.SemaphoreType.DMA,
            pltpu.SemaphoreType.DMA,
            pltpu.SemaphoreType.DMA,
            pltpu.SemaphoreType.DMA,
            pltpu.SemaphoreType.DMA,
            pltpu.SemaphoreType.DMA,
            pltpu.VMEM_SHARED((NA, D), jnp.float32),
        ],
    )
    return k(g, ei_prop, zz_rows)


_R = 2000


def _mm1_body(d0_ref, d1_ref, x_ref, w_ref, g_ref, dis_ref):
    deg = d0_ref[...] + d1_ref[...] + 1.0
    dis = lax.rsqrt(deg)
    h = jnp.dot(x_ref[...], w_ref[...], preferred_element_type=jnp.float32)
    g_ref[...] = h * dis
    dis_ref[...] = dis


def _mid_body(s_ref, g1_ref, dis_ref, b_ref, w_ref, g2_ref):
    dis = dis_ref[...]
    t = (s_ref[0] + s_ref[1] + g1_ref[...]) * dis + b_ref[...]
    t = jnp.maximum(t, 0.0)
    g2_ref[...] = jnp.dot(t, w_ref[...], preferred_element_type=jnp.float32) * dis


def _fin_body(s_ref, g2_ref, dis_ref, b_ref, o_ref):
    o_ref[...] = ((s_ref[0] + s_ref[1] + g2_ref[...]) * dis_ref[...]
                  + b_ref[...])


def _row_spec(width):
    return pl.BlockSpec((_R, width), lambda i: (i, 0))


def _s_spec():
    return pl.BlockSpec((NC, _R, D), lambda i: (0, i, 0))


def _full_spec(shape):
    return pl.BlockSpec(shape, lambda i: (0,) * len(shape))


def _mm1(d0, d1, x, W1):
    return pl.pallas_call(
        _mm1_body,
        grid=(N // _R,),
        in_specs=[_row_spec(1), _row_spec(1), _row_spec(D), _full_spec((D, D))],
        out_specs=[_row_spec(D), _row_spec(1)],
        out_shape=[jax.ShapeDtypeStruct((N, D), jnp.float32),
                   jax.ShapeDtypeStruct((N, 1), jnp.float32)],
    )(d0, d1, x, W1)


def _mid(s, g1, dis, b1, W2):
    return pl.pallas_call(
        _mid_body,
        grid=(N // _R,),
        in_specs=[_s_spec(), _row_spec(D), _row_spec(1),
                  _full_spec((1, D)), _full_spec((D, D))],
        out_specs=_row_spec(D),
        out_shape=jax.ShapeDtypeStruct((N, D), jnp.float32),
    )(s, g1, dis, b1, W2)


def _fin(s, g2, dis, b2):
    return pl.pallas_call(
        _fin_body,
        grid=(N // _R,),
        in_specs=[_s_spec(), _row_spec(D), _row_spec(1), _full_spec((1, D))],
        out_specs=_row_spec(D),
        out_shape=jax.ShapeDtypeStruct((N, D), jnp.float32),
    )(s, g2, dis, b2)


def kernel(x, edge_index, W1, b1, W2, b2):
    ei3 = edge_index.reshape(2, NW, EW)
    wk = jnp.arange(NW, dtype=jnp.int32)[:, None]
    pk = jnp.arange(PADW, dtype=jnp.int32)[None, :]
    dsrc = (wk * 977 + pk * 131) % N
    ddst = N + (wk * 37 + pk) % (NA - N)
    dummy = jnp.stack([dsrc, ddst])
    ei_prop = jnp.concatenate([ei3, dummy], axis=2).reshape(2, NW, NB, CB, C)
    ones_c = jnp.ones((C,), jnp.float32)
    zz_deg = jnp.zeros((DEGW,), jnp.float32)
    zz_rows = jnp.zeros((ROWS_PER_TILE, D), jnp.float32)

    deg = _degree(ei_prop, ones_c, zz_deg)
    g1, dis = _mm1(deg[0, :N, None], deg[1, :N, None], x, W1)

    s1 = _propagate(g1, ei_prop, zz_rows)
    g2 = _mid(s1, g1, dis, b1[None, :], W2)

    s2 = _propagate(g2, ei_prop, zz_rows)
    return _fin(s2, g2, dis, b2[None, :])
